# baseline (speedup 1.0000x reference)
import functools

import jax
import jax.numpy as jnp
from jax import lax
from jax.experimental import pallas as pl
from jax.experimental.pallas import tpu as pltpu
from jax.experimental.pallas import tpu_sc as plsc

NW = 8000
NS = 2000
E = 320000
IN_DIM = 128
OUT_DIM = 128
H = 8
HD = 16
FEAT = 16
FFN = 512
C = 64
NTILES = 32
DPT = 64
NSP = NTILES * DPT
A8N = NW * H + 8


def _t1a_body(w_ref, fcw_ref, attn_ref, z_ref, a_ref):
    wb = w_ref[...]
    cols = []
    farows = []
    for h in range(H):
        zh = jnp.dot(wb, fcw_ref[h], preferred_element_type=jnp.float32)
        cols.append(zh)
        fah = jnp.sum(fcw_ref[h] * attn_ref[h:h + 1, :HD], axis=1)
        farows.append(fah[None, :])
    z_ref[...] = jnp.concatenate(cols, axis=1)
    fa = jnp.concatenate(farows, axis=0)
    a_ref[...] = lax.dot_general(
        fa, wb, (((1,), (1,)), ((), ())),
        preferred_element_type=jnp.float32)


def _t1b_body(ef_ref, featw_ref, featb_ref, attn_ref, b_ref):
    ef = ef_ref[...]
    cols = []
    for h in range(H):
        aw2 = attn_ref[h:h + 1, HD:2 * HD]
        colh = jnp.sum(featw_ref[h] * aw2, axis=1)
        dh = jnp.sum(featb_ref[h:h + 1, :] * aw2, axis=1)
        bh = jnp.sum(ef * colh[None, :], axis=1, keepdims=True) + dh
        cols.append(bh)
    b = jnp.concatenate(cols, axis=1)
    b_ref[...] = jnp.concatenate([b, jnp.zeros_like(b)], axis=1)


def _sc_body(z_hbm, a8_hbm, b_hbm, src_hbm, dst_hbm, bounds_hbm,
             outz_hbm, outd_hbm,
             a8v, srcv, dstv, bv, zv, accv, denv, boundsv,
             gsem0, gsem1, psem0, psem1, ssem0, ssem1):
    cid = lax.axis_index("c")
    sid = lax.axis_index("s")
    wid = sid * 2 + cid

    pltpu.sync_copy(a8_hbm, a8v)
    pltpu.sync_copy(bounds_hbm.at[wid], boundsv)
    hlanes = jnp.bitwise_and(lax.iota(jnp.int32, 16), H - 1)

    zvec = jnp.zeros((16,), jnp.float32)

    def zero_acc(i, _):
        for j in range(IN_DIM // 16):
            accv[i, pl.ds(j * 16, 16)] = zvec
        denv[i, :] = zvec
        return 0

    lax.fori_loop(0, DPT, zero_acc, 0)

    binfo = boundsv[0, :]
    r0 = binfo[0]
    r1 = binfo[1]
    d0 = wid * DPT
    g0 = r0 // C
    ng = (r1 + (C - 1)) // C - g0
    lanes = lax.iota(jnp.int32, 16)

    gsem = (gsem0, gsem1)
    psem = (psem0, psem1)
    ssem = (ssem0, ssem1)

    def issue_src(j, k):
        e0 = (g0 + k) * C
        pltpu.async_copy(src_hbm.at[pl.ds(e0, C)], srcv.at[j], psem[j])

    def wait_src(j, k):
        e0 = (g0 + k) * C
        pltpu.make_async_copy(
            src_hbm.at[pl.ds(e0, C)], srcv.at[j], psem[j]).wait()

    def issue_rest(j, k):
        e0 = (g0 + k) * C
        pltpu.async_copy(z_hbm.at[srcv.at[j]], zv.at[j], gsem[j])
        pltpu.async_copy(dst_hbm.at[pl.ds(e0, C)], dstv.at[j], ssem[j])
        pltpu.async_copy(b_hbm.at[pl.ds(e0, C)], bv.at[j], ssem[j])

    def wait_rest(j, k):
        e0 = (g0 + k) * C
        pltpu.make_async_copy(z_hbm.at[srcv.at[j]], zv.at[j], gsem[j]).wait()
        pltpu.make_async_copy(
            dst_hbm.at[pl.ds(e0, C)], dstv.at[j], ssem[j]).wait()
        pltpu.make_async_copy(b_hbm.at[pl.ds(e0, C)], bv.at[j], ssem[j]).wait()

    @pl.when(ng > 0)
    def _prologue():
        issue_src(0, 0)
        wait_src(0, 0)
        issue_rest(0, 0)

        @pl.when(ng > 1)
        def _():
            issue_src(1, 1)

    def chunk_body(k, _):
        e0 = (g0 + k) * C
        jc = k % 2
        for j in range(2):
            jn = 1 - j

            @pl.when(jc == j)
            def _(j=j, jn=jn):
                wait_rest(j, k)

                @pl.when(k + 1 < ng)
                def _():
                    wait_src(jn, k + 1)
                    issue_rest(jn, k + 1)

                dstvj = dstv.at[j]
                bvj = bv.at[j]
                zvj = zv.at[j]

                def group_body(g, _):
                    src16 = srcv[j, pl.ds(g * 16, 16)]
                    dst16 = dstvj[pl.ds(g * 16, 16)]
                    for i in range(16):
                        row = g * 16 + i
                        absi = e0 + row
                        aidx = jnp.full((16,), src16[i], jnp.int32)
                        ag = plsc.load_gather(a8v, [hlanes, aidx])
                        x = ag + bvj[row, :]
                        e = jnp.maximum(x, 0.01 * x)
                        t = jnp.exp(e)
                        inr = jnp.logical_and(absi >= r0, absi < r1)
                        t = jnp.where(inr, t, zvec)
                        dloc = jnp.clip(dst16[i] - d0, 0, DPT - 1)
                        plsc.addupdate(denv.at[dloc, :], t)
                        for h in range(H):
                            zt = zvj[row, pl.ds(h * HD, HD)] * t[h]
                            plsc.addupdate(
                                accv.at[dloc, pl.ds(h * HD, HD)], zt)
                    return 0

                lax.fori_loop(0, C // 16, group_body, 0)

                @pl.when(k + 2 < ng)
                def _():
                    issue_src(j, k + 2)

        return 0

    lax.fori_loop(0, ng, chunk_body, 0)
    pltpu.sync_copy(accv, outz_hbm.at[pl.ds(d0, DPT)])
    pltpu.sync_copy(denv, outd_hbm.at[pl.ds(d0, DPT)])


@functools.cache
def _sc_agg():
    mesh = plsc.VectorSubcoreMesh(core_axis_name="c", subcore_axis_name="s",
                                  num_cores=2, num_subcores=16)
    return pl.kernel(
        _sc_body,
        out_type=[
            jax.ShapeDtypeStruct((NSP, IN_DIM), jnp.float32),
            jax.ShapeDtypeStruct((NSP, 16), jnp.float32),
        ],
        mesh=mesh,
        compiler_params=pltpu.CompilerParams(needs_layout_passes=False),
        scratch_types=[
            pltpu.VMEM((H, NW), jnp.float32),
            pltpu.VMEM((2, C), jnp.int32),
            pltpu.VMEM((2, C), jnp.int32),
            pltpu.VMEM((2, C, FEAT), jnp.float32),
            pltpu.VMEM((2, C, IN_DIM), jnp.float32),
            pltpu.VMEM((DPT, IN_DIM), jnp.float32),
            pltpu.VMEM((DPT, 16), jnp.float32),
            pltpu.VMEM((1, 16), jnp.int32),
            pltpu.SemaphoreType.DMA,
            pltpu.SemaphoreType.DMA,
            pltpu.SemaphoreType.DMA,
            pltpu.SemaphoreType.DMA,
            pltpu.SemaphoreType.DMA,
            pltpu.SemaphoreType.DMA,
        ],
    )


def _t2_body(pz_ref, pd_ref, s_ref, lng_ref, lnb_ref, w1_ref, b1_ref,
             w2_ref, b2_ref, out_ref):
    acc = pz_ref[:NS, :]
    den = pd_ref[:NS, :]
    hs = []
    for h in range(H):
        dh = den[:, h:h + 1]
        hm = acc[:, h * HD:(h + 1) * HD] / jnp.maximum(dh, 1e-16)
        hs.append(hm)
    hcat = jnp.concatenate(hs, axis=1)
    hv = jnp.where(hcat > 0.0, hcat,
                   jnp.exp(jnp.minimum(hcat, 0.0)) - 1.0) + s_ref[...]
    mu = jnp.mean(hv, axis=1, keepdims=True)
    var = jnp.mean((hv - mu) ** 2, axis=1, keepdims=True)
    ln = (hv - mu) / jnp.sqrt(var + 1e-6) * lng_ref[...] + lnb_ref[...]
    inter = jnp.dot(ln, w1_ref[...], preferred_element_type=jnp.float32)
    inter = inter + b1_ref[...]
    inter = inter * 0.5 * (1.0 + lax.erf(inter * (2.0 ** -0.5)))
    out = jnp.dot(inter, w2_ref[...], preferred_element_type=jnp.float32)
    out_ref[...] = out + b2_ref[...] + hv


def kernel(w, s, edge_src, edge_dst, edge_feat, fc_W, attn_W, feat_W,
           feat_b, ln_g, ln_b, w1, b1, w2, b2):
    bn = 8000
    z, a8 = pl.pallas_call(
        _t1a_body,
        grid=(NW // bn,),
        in_specs=[
            pl.BlockSpec((bn, IN_DIM), lambda i: (i, 0)),
            pl.BlockSpec((H, IN_DIM, HD), lambda i: (0, 0, 0)),
            pl.BlockSpec((H, 2 * HD), lambda i: (0, 0)),
        ],
        out_specs=[
            pl.BlockSpec((bn, IN_DIM), lambda i: (i, 0)),
            pl.BlockSpec((H, bn), lambda i: (0, i)),
        ],
        out_shape=[
            jax.ShapeDtypeStruct((NW, IN_DIM), jnp.float32),
            jax.ShapeDtypeStruct((H, NW), jnp.float32),
        ],
    )(w, fc_W, attn_W)

    be = 16000
    b16 = pl.pallas_call(
        _t1b_body,
        grid=(E // be,),
        in_specs=[
            pl.BlockSpec((be, FEAT), lambda i: (i, 0)),
            pl.BlockSpec((H, FEAT, HD), lambda i: (0, 0, 0)),
            pl.BlockSpec((H, HD), lambda i: (0, 0)),
            pl.BlockSpec((H, 2 * HD), lambda i: (0, 0)),
        ],
        out_specs=pl.BlockSpec((be, FEAT), lambda i: (i, 0)),
        out_shape=jax.ShapeDtypeStruct((E, FEAT), jnp.float32),
    )(edge_feat, feat_W, feat_b, attn_W)

    blk = (edge_dst // DPT)[:, None]
    counts = jnp.sum(
        (blk == jnp.arange(NTILES, dtype=jnp.int32)[None, :]).astype(
            jnp.int32), axis=0)
    r = jnp.concatenate([jnp.zeros((1,), jnp.int32),
                         jnp.cumsum(counts, dtype=jnp.int32)])
    bounds = jnp.zeros((NTILES, 1, 16), jnp.int32)
    bounds = bounds.at[:, 0, 0].set(r[:-1]).at[:, 0, 1].set(r[1:])
    pz, pd = _sc_agg()(z, a8, b16, edge_src, edge_dst, bounds)

    out = pl.pallas_call(
        _t2_body,
        out_shape=jax.ShapeDtypeStruct((NS, OUT_DIM), jnp.float32),
    )(pz, pd, s, ln_g.reshape(1, OUT_DIM), ln_b.reshape(1, OUT_DIM), w1,
      b1.reshape(1, FFN), w2, b2.reshape(1, OUT_DIM))
    return out


# speedup vs baseline: 1.3337x; 1.3337x over previous
import functools

import jax
import jax.numpy as jnp
from jax import lax
from jax.experimental import pallas as pl
from jax.experimental.pallas import tpu as pltpu
from jax.experimental.pallas import tpu_sc as plsc

NW = 8000
NS = 2000
E = 320000
IN_DIM = 128
OUT_DIM = 128
H = 8
HD = 16
FEAT = 16
FFN = 512
C = 64
NTILES = 32
DPT = 64
NSP = NTILES * DPT
A8N = NW * H + 8


def _t1a_body(w_ref, fcw_ref, attn_ref, z_ref, a_ref):
    wb = w_ref[...]
    cols = []
    farows = []
    for h in range(H):
        zh = jnp.dot(wb, fcw_ref[h], preferred_element_type=jnp.float32)
        cols.append(zh)
        fah = jnp.sum(fcw_ref[h] * attn_ref[h:h + 1, :HD], axis=1)
        farows.append(fah[None, :])
    z_ref[...] = jnp.concatenate(cols, axis=1)
    fa = jnp.concatenate(farows, axis=0)
    a_ref[...] = lax.dot_general(
        fa, wb, (((1,), (1,)), ((), ())),
        preferred_element_type=jnp.float32)


def _t1b_body(efw_ref, featw_ref, featb_ref, attn_ref, b_ref):
    cols = []
    drow = []
    z16 = jnp.zeros((HD, H), jnp.float32)
    for h in range(H):
        aw2 = attn_ref[h:h + 1, HD:2 * HD]
        cols.append(jnp.sum(featw_ref[h] * aw2, axis=1, keepdims=True))
        drow.append(jnp.sum(featb_ref[h:h + 1, :] * aw2, axis=1,
                            keepdims=True))
    c16 = jnp.concatenate(cols + [z16[:, :H]], axis=1)
    d16 = jnp.concatenate(drow + [z16[:1, :H]], axis=1)
    zb = jnp.zeros((HD, HD), jnp.float32)
    brows = []
    for p in range(H):
        brows.append(jnp.concatenate(
            [zb] * p + [c16] + [zb] * (H - 1 - p), axis=1))
    bigc = jnp.concatenate(brows, axis=0)
    d128 = jnp.concatenate([d16] * H, axis=1)
    b_ref[...] = jnp.dot(efw_ref[...], bigc,
                         preferred_element_type=jnp.float32) + d128


def _sc_body(z_hbm, a8_hbm, b_hbm, src_hbm, dst_hbm, bounds_hbm,
             outz_hbm, outd_hbm,
             a8v, srcv, dstv, bv, zv, accv, denv, boundsv,
             gsem0, gsem1, psem0, psem1, ssem0, ssem1):
    cid = lax.axis_index("c")
    sid = lax.axis_index("s")
    wid = sid * 2 + cid

    pltpu.sync_copy(a8_hbm, a8v)
    pltpu.sync_copy(bounds_hbm.at[wid], boundsv)
    hlanes = jnp.bitwise_and(lax.iota(jnp.int32, 16), H - 1)

    zvec = jnp.zeros((16,), jnp.float32)

    def zero_acc(i, _):
        for j in range(IN_DIM // 16):
            accv[i, pl.ds(j * 16, 16)] = zvec
        denv[i, :] = zvec
        return 0

    lax.fori_loop(0, DPT, zero_acc, 0)

    binfo = boundsv[0, :]
    r0 = binfo[0]
    r1 = binfo[1]
    d0 = wid * DPT
    g0 = r0 // C
    ng = (r1 + (C - 1)) // C - g0
    lanes = lax.iota(jnp.int32, 16)

    gsem = (gsem0, gsem1)
    psem = (psem0, psem1)
    ssem = (ssem0, ssem1)

    def issue_src(j, k):
        e0 = (g0 + k) * C
        pltpu.async_copy(src_hbm.at[pl.ds(e0, C)], srcv.at[j], psem[j])

    def wait_src(j, k):
        e0 = (g0 + k) * C
        pltpu.make_async_copy(
            src_hbm.at[pl.ds(e0, C)], srcv.at[j], psem[j]).wait()

    def issue_rest(j, k):
        e0 = (g0 + k) * C
        e8 = pl.multiple_of((g0 + k) * (C // 8), 8)
        pltpu.async_copy(z_hbm.at[srcv.at[j]], zv.at[j], gsem[j])
        pltpu.async_copy(dst_hbm.at[pl.ds(e0, C)], dstv.at[j], ssem[j])
        pltpu.async_copy(b_hbm.at[pl.ds(e8, C // 8)], bv.at[j], ssem[j])

    def wait_rest(j, k):
        e0 = (g0 + k) * C
        e8 = pl.multiple_of((g0 + k) * (C // 8), 8)
        pltpu.make_async_copy(z_hbm.at[srcv.at[j]], zv.at[j], gsem[j]).wait()
        pltpu.make_async_copy(
            dst_hbm.at[pl.ds(e0, C)], dstv.at[j], ssem[j]).wait()
        pltpu.make_async_copy(
            b_hbm.at[pl.ds(e8, C // 8)], bv.at[j], ssem[j]).wait()

    @pl.when(ng > 0)
    def _prologue():
        issue_src(0, 0)
        wait_src(0, 0)
        issue_rest(0, 0)

        @pl.when(ng > 1)
        def _():
            issue_src(1, 1)

    def chunk_body(k, _):
        e0 = (g0 + k) * C
        jc = k % 2
        for j in range(2):
            jn = 1 - j

            @pl.when(jc == j)
            def _(j=j, jn=jn):
                wait_rest(j, k)

                @pl.when(k + 1 < ng)
                def _():
                    wait_src(jn, k + 1)
                    issue_rest(jn, k + 1)

                dstvj = dstv.at[j]
                bvj = bv.at[j]
                zvj = zv.at[j]

                def group_body(g, _):
                    src16 = srcv[j, pl.ds(g * 16, 16)]
                    dst16 = dstvj[pl.ds(g * 16, 16)]
                    for i in range(16):
                        row = g * 16 + i
                        absi = e0 + row
                        aidx = jnp.full((16,), src16[i], jnp.int32)
                        ag = plsc.load_gather(a8v, [hlanes, aidx])
                        x = ag + bvj[row >> 3, pl.ds((row & 7) * 16, 16)]
                        e = jnp.maximum(x, 0.01 * x)
                        t = jnp.exp(e)
                        inr = jnp.logical_and(absi >= r0, absi < r1)
                        t = jnp.where(inr, t, zvec)
                        dloc = jnp.clip(dst16[i] - d0, 0, DPT - 1)
                        plsc.addupdate(denv.at[dloc, :], t)
                        for h in range(H):
                            zt = zvj[row, pl.ds(h * HD, HD)] * t[h]
                            plsc.addupdate(
                                accv.at[dloc, pl.ds(h * HD, HD)], zt)
                    return 0

                lax.fori_loop(0, C // 16, group_body, 0)

                @pl.when(k + 2 < ng)
                def _():
                    issue_src(j, k + 2)

        return 0

    lax.fori_loop(0, ng, chunk_body, 0)
    pltpu.sync_copy(accv, outz_hbm.at[pl.ds(d0, DPT)])
    pltpu.sync_copy(denv, outd_hbm.at[pl.ds(d0, DPT)])


@functools.cache
def _sc_agg():
    mesh = plsc.VectorSubcoreMesh(core_axis_name="c", subcore_axis_name="s",
                                  num_cores=2, num_subcores=16)
    return pl.kernel(
        _sc_body,
        out_type=[
            jax.ShapeDtypeStruct((NSP, IN_DIM), jnp.float32),
            jax.ShapeDtypeStruct((NSP, 16), jnp.float32),
        ],
        mesh=mesh,
        compiler_params=pltpu.CompilerParams(needs_layout_passes=False),
        scratch_types=[
            pltpu.VMEM((H, NW), jnp.float32),
            pltpu.VMEM((2, C), jnp.int32),
            pltpu.VMEM((2, C), jnp.int32),
            pltpu.VMEM((2, C // 8, 128), jnp.float32),
            pltpu.VMEM((2, C, IN_DIM), jnp.float32),
            pltpu.VMEM((DPT, IN_DIM), jnp.float32),
            pltpu.VMEM((DPT, 16), jnp.float32),
            pltpu.VMEM((1, 16), jnp.int32),
            pltpu.SemaphoreType.DMA,
            pltpu.SemaphoreType.DMA,
            pltpu.SemaphoreType.DMA,
            pltpu.SemaphoreType.DMA,
            pltpu.SemaphoreType.DMA,
            pltpu.SemaphoreType.DMA,
        ],
    )


def _t2_body(pz_ref, pd_ref, s_ref, lng_ref, lnb_ref, w1_ref, b1_ref,
             w2_ref, b2_ref, out_ref):
    acc = pz_ref[:NS, :]
    den = pd_ref[:NS, :]
    hs = []
    for h in range(H):
        dh = den[:, h:h + 1]
        hm = acc[:, h * HD:(h + 1) * HD] / jnp.maximum(dh, 1e-16)
        hs.append(hm)
    hcat = jnp.concatenate(hs, axis=1)
    hv = jnp.where(hcat > 0.0, hcat,
                   jnp.exp(jnp.minimum(hcat, 0.0)) - 1.0) + s_ref[...]
    mu = jnp.mean(hv, axis=1, keepdims=True)
    var = jnp.mean((hv - mu) ** 2, axis=1, keepdims=True)
    ln = (hv - mu) / jnp.sqrt(var + 1e-6) * lng_ref[...] + lnb_ref[...]
    inter = jnp.dot(ln, w1_ref[...], preferred_element_type=jnp.float32)
    inter = inter + b1_ref[...]
    inter = inter * 0.5 * (1.0 + lax.erf(inter * (2.0 ** -0.5)))
    out = jnp.dot(inter, w2_ref[...], preferred_element_type=jnp.float32)
    out_ref[...] = out + b2_ref[...] + hv


def kernel(w, s, edge_src, edge_dst, edge_feat, fc_W, attn_W, feat_W,
           feat_b, ln_g, ln_b, w1, b1, w2, b2):
    bn = 8000
    z, a8 = pl.pallas_call(
        _t1a_body,
        grid=(NW // bn,),
        in_specs=[
            pl.BlockSpec((bn, IN_DIM), lambda i: (i, 0)),
            pl.BlockSpec((H, IN_DIM, HD), lambda i: (0, 0, 0)),
            pl.BlockSpec((H, 2 * HD), lambda i: (0, 0)),
        ],
        out_specs=[
            pl.BlockSpec((bn, IN_DIM), lambda i: (i, 0)),
            pl.BlockSpec((H, bn), lambda i: (0, i)),
        ],
        out_shape=[
            jax.ShapeDtypeStruct((NW, IN_DIM), jnp.float32),
            jax.ShapeDtypeStruct((H, NW), jnp.float32),
        ],
    )(w, fc_W, attn_W)

    be = 16000
    efw = edge_feat.reshape(E // 8, 128)
    b16 = pl.pallas_call(
        _t1b_body,
        grid=(E // be,),
        in_specs=[
            pl.BlockSpec((be // 8, 128), lambda i: (i, 0)),
            pl.BlockSpec((H, FEAT, HD), lambda i: (0, 0, 0)),
            pl.BlockSpec((H, HD), lambda i: (0, 0)),
            pl.BlockSpec((H, 2 * HD), lambda i: (0, 0)),
        ],
        out_specs=pl.BlockSpec((be // 8, 128), lambda i: (i, 0)),
        out_shape=jax.ShapeDtypeStruct((E // 8, 128), jnp.float32),
    )(efw, feat_W, feat_b, attn_W)

    blk = (edge_dst // DPT)[:, None]
    counts = jnp.sum(
        (blk == jnp.arange(NTILES, dtype=jnp.int32)[None, :]).astype(
            jnp.int32), axis=0)
    r = jnp.concatenate([jnp.zeros((1,), jnp.int32),
                         jnp.cumsum(counts, dtype=jnp.int32)])
    bounds = jnp.zeros((NTILES, 1, 16), jnp.int32)
    bounds = bounds.at[:, 0, 0].set(r[:-1]).at[:, 0, 1].set(r[1:])
    pz, pd = _sc_agg()(z, a8, b16, edge_src, edge_dst, bounds)

    out = pl.pallas_call(
        _t2_body,
        out_shape=jax.ShapeDtypeStruct((NS, OUT_DIM), jnp.float32),
    )(pz, pd, s, ln_g.reshape(1, OUT_DIM), ln_b.reshape(1, OUT_DIM), w1,
      b1.reshape(1, FFN), w2, b2.reshape(1, OUT_DIM))
    return out
